# baseline (speedup 1.0000x reference)
import jax
import jax.numpy as jnp
from jax.experimental import pallas as pl
from jax.experimental.pallas import tpu as pltpu

_BM = 200
_NBUF = 4


def _mm_kernel(adj_hbm, h_ref, out_ref, buf, sems):
    i = pl.program_id(0)
    nsteps = pl.num_programs(0)

    def copy(block_idx, slot):
        return pltpu.make_async_copy(
            adj_hbm.at[pl.ds(block_idx * _BM, _BM), :],
            buf.at[slot],
            sems.at[slot],
        )

    @pl.when(i == 0)
    def _prologue():
        for b in range(_NBUF):
            copy(b, b).start()

    slot = jax.lax.rem(i, _NBUF)
    copy(i, slot).wait()
    out_ref[...] = jnp.dot(buf[slot], h_ref[...],
                           preferred_element_type=jnp.float32)

    @pl.when(i + _NBUF < nsteps)
    def _refill():
        copy(i + _NBUF, slot).start()


def kernel(adj, h):
    n, k = adj.shape
    d = h.shape[1]
    grid = (n // _BM,)
    return pl.pallas_call(
        _mm_kernel,
        grid=grid,
        in_specs=[
            pl.BlockSpec(memory_space=pltpu.MemorySpace.HBM),
            pl.BlockSpec((k, d), lambda i: (0, 0)),
        ],
        out_specs=pl.BlockSpec((_BM, d), lambda i: (i, 0)),
        out_shape=jax.ShapeDtypeStruct((n, d), jnp.float32),
        scratch_shapes=[
            pltpu.VMEM((_NBUF, _BM, k), jnp.float32),
            pltpu.SemaphoreType.DMA((_NBUF,)),
        ],
    )(adj, h)


# speedup vs baseline: 1.0109x; 1.0109x over previous
import jax
import jax.numpy as jnp
from jax.experimental import pallas as pl
from jax.experimental.pallas import tpu as pltpu

_BM = 200
_NBUF = 4


def _mm_kernel(adj_hbm, h_ref, out_ref, buf, sems):
    i = pl.program_id(0)
    nsteps = pl.num_programs(0)

    def copy(block_idx, slot):
        return pltpu.make_async_copy(
            adj_hbm.at[pl.ds(block_idx * _BM, _BM), :],
            buf.at[slot],
            sems.at[slot],
        )

    @pl.when(i == 0)
    def _prologue():
        for b in range(_NBUF - 1):
            copy(b, b).start()

    slot = jax.lax.rem(i, _NBUF)
    copy(i, slot).wait()

    nxt = i + _NBUF - 1
    @pl.when(nxt < nsteps)
    def _refill():
        copy(nxt, jax.lax.rem(nxt, _NBUF)).start()

    out_ref[...] = jnp.dot(buf[slot], h_ref[...],
                           preferred_element_type=jnp.float32)


def kernel(adj, h):
    n, k = adj.shape
    d = h.shape[1]
    grid = (n // _BM,)
    return pl.pallas_call(
        _mm_kernel,
        grid=grid,
        in_specs=[
            pl.BlockSpec(memory_space=pltpu.MemorySpace.HBM),
            pl.BlockSpec((k, d), lambda i: (0, 0)),
        ],
        out_specs=pl.BlockSpec((_BM, d), lambda i: (i, 0)),
        out_shape=jax.ShapeDtypeStruct((n, d), jnp.float32),
        scratch_shapes=[
            pltpu.VMEM((_NBUF, _BM, k), jnp.float32),
            pltpu.SemaphoreType.DMA((_NBUF,)),
        ],
    )(adj, h)
